# baseline (speedup 1.0000x reference)
import functools

import jax
import jax.numpy as jnp
import numpy as np
from jax import lax
from jax.experimental import pallas as pl
from jax.experimental.pallas import tpu as pltpu


def _body(adj_ref, mask_ref, emb_ref, ret_ref, cache_ref, lat_ref, *, bn, n):
    p = pl.program_id(0)
    i = pl.program_id(1)

    @pl.when(p == 0)
    def _phase0():
        w = mask_ref[...]
        h = adj_ref.shape[0]
        wrep = jnp.concatenate(
            [jnp.broadcast_to(w[0:1], (32, bn)),
             jnp.broadcast_to(w[1:2], (32, bn))], axis=0)
        sham = lax.rem(
            lax.broadcasted_iota(jnp.uint32, (h, bn), 0), jnp.uint32(32))
        bits = jnp.bitwise_and(jnp.right_shift(wrep, sham), jnp.uint32(1))
        row = lax.broadcasted_iota(jnp.int32, (h, bn), 1) + i * bn
        keep = jnp.logical_and(bits != 0, row < n)
        ad_t = jnp.where(keep, adj_ref[...] * 2.0, 0.0).astype(jnp.bfloat16)
        cache_ref[pl.ds(i, 1), :, :] = ad_t[None]
        erow = lax.broadcasted_iota(jnp.int32, emb_ref.shape, 0) + i * bn
        emb = jnp.where(erow < n, emb_ref[...], 0.0).astype(jnp.bfloat16)
        partial = lax.dot_general(
            ad_t, emb, (((1,), (0,)), ((), ())),
            preferred_element_type=jnp.float32)

        @pl.when(i == 0)
        def _():
            lat_ref[...] = partial

        @pl.when(i > 0)
        def _():
            lat_ref[...] += partial

    @pl.when(p == 1)
    def _phase1():
        ad_t = cache_ref[pl.ds(i, 1), :, :][0]
        ret_ref[...] = lax.dot_general(
            ad_t, lat_ref[...].astype(jnp.bfloat16), (((0,), (0,)), ((), ())),
            preferred_element_type=jnp.float32)


_MASK_CACHE = {}


def _threefry2x32(k0, k1, x0, x1):
    ks0 = np.uint32(k0); ks1 = np.uint32(k1)
    ks2 = np.uint32(ks0 ^ ks1 ^ np.uint32(0x1BD11BDA))
    ks = (ks0, ks1, ks2)
    rot = ((13, 15, 26, 6), (17, 29, 16, 24))
    x0 = (x0 + ks0).astype(np.uint32)
    x1 = (x1 + ks1).astype(np.uint32)
    for j in range(5):
        for rr in rot[j % 2]:
            x0 = (x0 + x1).astype(np.uint32)
            x1 = ((x1 << np.uint32(rr)) | (x1 >> np.uint32(32 - rr))).astype(np.uint32)
            x1 = x1 ^ x0
        x0 = (x0 + ks[(j + 1) % 3]).astype(np.uint32)
        x1 = (x1 + ks[(j + 2) % 3] + np.uint32(j + 1)).astype(np.uint32)
    return x0, x1


def _bernoulli_half_mask(seed, shape):
    size = int(np.prod(shape))
    i = np.arange(size, dtype=np.uint64)
    o0, o1 = _threefry2x32(np.uint32(seed >> 32), np.uint32(seed & 0xFFFFFFFF),
                           (i >> np.uint64(32)).astype(np.uint32),
                           i.astype(np.uint32))
    return ((o0 ^ o1) < np.uint32(0x80000000)).reshape(shape)


def _keep_mask_packed(shape):
    if shape not in _MASK_CACHE:
        m = _bernoulli_half_mask(42, shape).astype(np.uint32)
        n, h = shape
        packed = (m.reshape(n, h // 32, 32)
                  << np.arange(32, dtype=np.uint32)[None, None, :]
                  ).sum(axis=2, dtype=np.uint32)
        _MASK_CACHE[shape] = np.ascontiguousarray(packed.T)
    return _MASK_CACHE[shape]


def kernel(adj, embeds):
    n, h = adj.shape
    d = embeds.shape[1]
    bn = 11136
    steps = -(-n // bn)
    mask = _keep_mask_packed((n, h))

    adj_t = adj.T

    grid = (2, steps)
    out = pl.pallas_call(
        functools.partial(_body, bn=bn, n=n),
        grid=grid,
        in_specs=[
            pl.BlockSpec((h, bn), lambda p, i, s=steps: (0, jnp.where(p == 0, i, s - 1))),
            pl.BlockSpec((h // 32, bn),
                         lambda p, i, s=steps: (0, jnp.where(p == 0, i, s - 1))),
            pl.BlockSpec((bn, d), lambda p, i, s=steps: (jnp.where(p == 0, i, s - 1), 0)),
        ],
        out_specs=pl.BlockSpec((bn, d), lambda p, i: (jnp.where(p == 0, 0, i), 0)),
        out_shape=jax.ShapeDtypeStruct((n, d), jnp.float32),
        scratch_shapes=[
            pltpu.VMEM((steps, h, bn), jnp.bfloat16),
            pltpu.VMEM((h, d), jnp.float32),
        ],
    )(adj_t, mask, embeds)
    return out


# speedup vs baseline: 1.0116x; 1.0116x over previous
import functools

import jax
import jax.numpy as jnp
import numpy as np
from jax import lax
from jax.experimental import pallas as pl
from jax.experimental.pallas import tpu as pltpu


def _body(adj_ref, mask_ref, emb_ref, ret_ref, cache_ref, lat_ref, *, bn, n):
    p = pl.program_id(0)
    i = pl.program_id(1)

    @pl.when(p == 0)
    def _phase0():
        w = mask_ref[...]
        h = adj_ref.shape[0]
        wrep = jnp.concatenate(
            [jnp.broadcast_to(w[0:1], (32, bn)),
             jnp.broadcast_to(w[1:2], (32, bn))], axis=0)
        sham = lax.rem(
            lax.broadcasted_iota(jnp.uint32, (h, bn), 0), jnp.uint32(32))
        bits = jnp.bitwise_and(jnp.right_shift(wrep, sham), jnp.uint32(1))
        row = lax.broadcasted_iota(jnp.int32, (h, bn), 1) + i * bn
        keep = jnp.logical_and(bits != 0, row < n)
        ad_t = jnp.where(keep, adj_ref[...] * 2.0, 0.0).astype(jnp.bfloat16)
        cache_ref[pl.ds(i, 1), :, :] = ad_t[None]
        erow = lax.broadcasted_iota(jnp.int32, emb_ref.shape, 0) + i * bn
        emb = jnp.where(erow < n, emb_ref[...], 0.0).astype(jnp.bfloat16)
        partial = lax.dot_general(
            ad_t, emb, (((1,), (0,)), ((), ())),
            preferred_element_type=jnp.float32)

        @pl.when(i == 0)
        def _():
            lat_ref[...] = partial

        @pl.when(i > 0)
        def _():
            lat_ref[...] += partial

    @pl.when(p == 1)
    def _phase1():
        ad_t = cache_ref[pl.ds(i, 1), :, :][0]
        ret_ref[...] = lax.dot_general(
            ad_t, lat_ref[...].astype(jnp.bfloat16), (((0,), (0,)), ((), ())),
            preferred_element_type=jnp.float32)


_MASK_CACHE = {}


def _threefry2x32(k0, k1, x0, x1):
    ks0 = np.uint32(k0); ks1 = np.uint32(k1)
    ks2 = np.uint32(ks0 ^ ks1 ^ np.uint32(0x1BD11BDA))
    ks = (ks0, ks1, ks2)
    rot = ((13, 15, 26, 6), (17, 29, 16, 24))
    x0 = (x0 + ks0).astype(np.uint32)
    x1 = (x1 + ks1).astype(np.uint32)
    for j in range(5):
        for rr in rot[j % 2]:
            x0 = (x0 + x1).astype(np.uint32)
            x1 = ((x1 << np.uint32(rr)) | (x1 >> np.uint32(32 - rr))).astype(np.uint32)
            x1 = x1 ^ x0
        x0 = (x0 + ks[(j + 1) % 3]).astype(np.uint32)
        x1 = (x1 + ks[(j + 2) % 3] + np.uint32(j + 1)).astype(np.uint32)
    return x0, x1


def _bernoulli_half_mask(seed, shape):
    size = int(np.prod(shape))
    i = np.arange(size, dtype=np.uint64)
    o0, o1 = _threefry2x32(np.uint32(seed >> 32), np.uint32(seed & 0xFFFFFFFF),
                           (i >> np.uint64(32)).astype(np.uint32),
                           i.astype(np.uint32))
    return ((o0 ^ o1) < np.uint32(0x80000000)).reshape(shape)


def _keep_mask_packed(shape):
    if shape not in _MASK_CACHE:
        m = _bernoulli_half_mask(42, shape).astype(np.uint32)
        n, h = shape
        packed = (m.reshape(n, h // 32, 32)
                  << np.arange(32, dtype=np.uint32)[None, None, :]
                  ).sum(axis=2, dtype=np.uint32)
        _MASK_CACHE[shape] = np.ascontiguousarray(packed.T)
    return _MASK_CACHE[shape]


def kernel(adj, embeds):
    n, h = adj.shape
    d = embeds.shape[1]
    bn = 12544
    steps = -(-n // bn)
    mask = _keep_mask_packed((n, h))

    adj_t = adj.T

    grid = (2, steps)
    out = pl.pallas_call(
        functools.partial(_body, bn=bn, n=n),
        grid=grid,
        in_specs=[
            pl.BlockSpec((h, bn), lambda p, i, s=steps: (0, jnp.where(p == 0, i, s - 1))),
            pl.BlockSpec((h // 32, bn),
                         lambda p, i, s=steps: (0, jnp.where(p == 0, i, s - 1))),
            pl.BlockSpec((bn, d), lambda p, i, s=steps: (jnp.where(p == 0, i, s - 1), 0)),
        ],
        out_specs=pl.BlockSpec((bn, d), lambda p, i: (jnp.where(p == 0, 0, i), 0)),
        out_shape=jax.ShapeDtypeStruct((n, d), jnp.float32),
        scratch_shapes=[
            pltpu.VMEM((steps, h, bn), jnp.bfloat16),
            pltpu.VMEM((h, d), jnp.float32),
        ],
    )(adj_t, mask, embeds)
    return out
